# baseline (speedup 1.0000x reference)
import functools

import jax
import jax.numpy as jnp
from jax import lax
from jax.experimental import pallas as pl
from jax.experimental.pallas import tpu as pltpu
from jax.experimental.pallas import tpu_sc as plsc

NUM_ROWS = 129
EMBED_D = 64
BATCH = 4096
HIST = 200

_NC = 2
_NS = 16
_NW = _NC * _NS
_BPW = BATCH // _NW
_L = 16
_TSTRIDE = 144
_HC = 8
_NHC = HIST // _HC

_mesh = plsc.VectorSubcoreMesh(core_axis_name="c", subcore_axis_name="s")


@functools.partial(
    pl.kernel,
    mesh=_mesh,
    compiler_params=pltpu.CompilerParams(needs_layout_passes=False),
    out_type=jax.ShapeDtypeStruct((HIST, EMBED_D, BATCH), jnp.float32),
    scratch_types=[
        pltpu.VMEM((EMBED_D, NUM_ROWS), jnp.float32),
        pltpu.VMEM((EMBED_D * _TSTRIDE,), jnp.float32),
        pltpu.VMEM((_HC, _BPW), jnp.int32),
        pltpu.VMEM((_HC, _BPW), jnp.int32),
        pltpu.VMEM((EMBED_D, _BPW), jnp.float32),
        pltpu.VMEM((EMBED_D, _BPW), jnp.float32),
        pltpu.SemaphoreType.DMA,
        pltpu.SemaphoreType.DMA,
        pltpu.SemaphoreType.DMA,
        pltpu.SemaphoreType.DMA,
    ],
)
def _embed_gather(idx_hbm, table_hbm, out_hbm, table2d, table_f, idx0, idx1,
                  slab0, slab1, si0, si1, so0, so1):
    wid = lax.axis_index("s") * _NC + lax.axis_index("c")
    b0 = wid * _BPW

    pltpu.sync_copy(table_hbm, table2d)
    pltpu.async_copy(idx_hbm.at[pl.ds(0, _HC), pl.ds(b0, _BPW)], idx0, si0)
    pltpu.async_copy(idx_hbm.at[pl.ds(_HC, _HC), pl.ds(b0, _BPW)], idx1, si1)

    def flat(d, carry):
        for k in range(NUM_ROWS // _L):
            v = table2d[d, pl.ds(k * _L, _L)]
            table_f[pl.ds(d * _TSTRIDE + k * _L, _L)] = v
        v = table2d[d, pl.ds(NUM_ROWS - _L, _L)]
        table_f[pl.ds(d * _TSTRIDE + NUM_ROWS - _L, _L)] = v
        return carry

    lax.fori_loop(0, EMBED_D, flat, 0)

    def slab_compute(iv_ref, hh, rv):
        for s in range(_BPW // _L):
            g = iv_ref[hh, pl.ds(s * _L, _L)]
            for d in range(EMBED_D):
                vals = plsc.load_gather(table_f, [g + (d * _TSTRIDE)])
                rv[d, pl.ds(s * _L, _L)] = vals

    def chunk(hc, carry):
        for p, (iv_ref, si) in enumerate(((idx0, si0), (idx1, si1))):
            @pl.when(hc % 2 == p)
            def _run():
                pltpu.make_async_copy(
                    idx_hbm.at[pl.ds(0, _HC), pl.ds(b0, _BPW)], iv_ref,
                    si).wait()

                def hpair(q, carry2):
                    for par, (rv, so) in enumerate(((slab0, so0),
                                                    (slab1, so1))):
                        hh = 2 * q + par
                        h = hc * _HC + hh

                        @pl.when(h >= 2)
                        def _wait_out():
                            pltpu.make_async_copy(
                                rv, out_hbm.at[0, :, pl.ds(b0, _BPW)],
                                so).wait()

                        slab_compute(iv_ref, hh, rv)
                        pltpu.async_copy(
                            rv, out_hbm.at[h, :, pl.ds(b0, _BPW)], so)
                    return carry2

                lax.fori_loop(0, _HC // 2, hpair, 0)

                @pl.when(hc + 2 < _NHC)
                def _prefetch():
                    pltpu.async_copy(
                        idx_hbm.at[pl.ds((hc + 2) * _HC, _HC),
                                   pl.ds(b0, _BPW)], iv_ref, si)
        return carry

    lax.fori_loop(0, _NHC, chunk, 0)

    pltpu.make_async_copy(slab0, out_hbm.at[0, :, pl.ds(b0, _BPW)], so0).wait()
    pltpu.make_async_copy(slab1, out_hbm.at[0, :, pl.ds(b0, _BPW)], so1).wait()


def kernel(genres, table):
    gt = genres.astype(jnp.int32).T
    tt = table.T
    out_t = _embed_gather(gt, tt)
    return out_t.transpose(2, 0, 1)
